# baseline (device time: 20216 ns/iter reference)
import jax
import jax.numpy as jnp
from jax import lax
from jax.experimental import pallas as pl
from jax.experimental.pallas import tpu as pltpu

N_DEV = 32


def kernel(x):
    m_per, n = x.shape

    def body(x_ref, out_ref, xv_ref, recv_ref, send_sems, recv_sems, copy_sems):
        my_pos = lax.axis_index("i")
        barrier_sem = pltpu.get_barrier_semaphore()
        rounds = [(1, 2, 3), (4, 8, 12), (16,)]

        n_chunks = 4
        mc = m_per // n_chunks
        rows = lax.broadcasted_iota(jnp.int32, (mc, n), 0)

        def signal_round(offs):
            for o in offs:
                pl.semaphore_signal(
                    barrier_sem,
                    inc=1,
                    device_id=((my_pos + o) % N_DEV,),
                    device_id_type=pl.DeviceIdType.MESH,
                )

        signal_round(rounds[0])
        copies = []
        for c in range(n_chunks):
            sl = pl.ds(c * mc, mc)
            cp = pltpu.make_async_copy(
                x_ref.at[sl, :], xv_ref.at[sl, :], copy_sems.at[c]
            )
            cp.start()
            copies.append(cp)

        def compute_chunk(c, acc):
            copies[c].wait()
            xv = xv_ref[pl.ds(c * mc, mc), :]
            vc = jnp.max(xv, axis=0)
            masked = jnp.where(xv == vc[None, :], rows, mc)
            ic = jnp.min(masked, axis=0) + (c * mc + my_pos * m_per)
            if acc is None:
                return vc, ic
            v, i = acc
            take = vc > v
            return jnp.where(take, vc, v), jnp.where(take, ic, i)

        def send_to(offsets_behind):
            out = []
            for o in offsets_behind:
                peer = (my_pos - o) % N_DEV
                rdma = pltpu.make_async_remote_copy(
                    src_ref=recv_ref.at[my_pos],
                    dst_ref=recv_ref.at[my_pos],
                    send_sem=send_sems.at[o],
                    recv_sem=recv_sems.at[my_pos],
                    device_id=(peer,),
                    device_id_type=pl.DeviceIdType.MESH,
                )
                rdma.start()
                out.append(rdma)
            return out

        acc = compute_chunk(0, None)
        acc = compute_chunk(1, acc)
        pl.semaphore_wait(barrier_sem, 3)
        signal_round(rounds[1])
        acc = compute_chunk(2, acc)
        acc = compute_chunk(3, acc)
        recv_ref[my_pos, 0, :] = acc[0]
        recv_ref[my_pos, 1, :] = acc[1].astype(jnp.float32)
        pl.semaphore_wait(barrier_sem, 3)
        signal_round(rounds[2])
        sends = send_to(range(1, 16))
        pl.semaphore_wait(barrier_sem, 1)
        sends += send_to(range(16, N_DEV))

        for p in range(1, N_DEV):
            src = my_pos ^ p
            recv = pltpu.make_async_remote_copy(
                src_ref=recv_ref.at[src],
                dst_ref=recv_ref.at[src],
                send_sem=send_sems.at[p],
                recv_sem=recv_sems.at[src],
                device_id=(src,),
                device_id_type=pl.DeviceIdType.MESH,
            )
            recv.wait_recv()

        vals = recv_ref[:, 0, :]
        idxs = recv_ref[:, 1, :]
        m = jnp.max(vals, axis=0)
        gi = jnp.min(
            jnp.where(vals == m[None, :], idxs, float(N_DEV * m_per)), axis=0
        )
        out_ref[0, :] = m
        out_ref[1, :] = gi

        for rdma in sends:
            rdma.wait_send()

    return pl.pallas_call(
        body,
        out_shape=jax.ShapeDtypeStruct((2, n), jnp.float32),
        in_specs=[pl.BlockSpec(memory_space=pl.ANY)],
        out_specs=pl.BlockSpec(memory_space=pltpu.VMEM),
        scratch_shapes=[
            pltpu.VMEM((m_per, n), jnp.float32),
            pltpu.VMEM((N_DEV, 2, n), jnp.float32),
            pltpu.SemaphoreType.DMA((N_DEV,)),
            pltpu.SemaphoreType.DMA((N_DEV,)),
            pltpu.SemaphoreType.DMA((4,)),
        ],
        compiler_params=pltpu.CompilerParams(collective_id=0),
    )(x)


# device time: 18107 ns/iter; 1.1165x vs baseline; 1.1165x over previous
import jax
import jax.numpy as jnp
from jax import lax
from jax.experimental import pallas as pl
from jax.experimental.pallas import tpu as pltpu

N_DEV = 32


def kernel(x):
    m_per, n = x.shape

    def body(x_ref, out_ref, recv_ref, send_sems, recv_sems):
        my_pos = lax.axis_index("i")

        barrier_sem = pltpu.get_barrier_semaphore()
        for o in range(1, N_DEV):
            pl.semaphore_signal(
                barrier_sem,
                inc=1,
                device_id=((my_pos + o) % N_DEV,),
                device_id_type=pl.DeviceIdType.MESH,
            )

        xv = x_ref[:, :]
        val = jnp.max(xv, axis=0)
        rows = lax.broadcasted_iota(jnp.int32, (m_per, n), 0)
        masked = jnp.where(xv == val[None, :], rows, m_per)
        idx = jnp.min(masked, axis=0) + my_pos * m_per
        recv_ref[my_pos, 0, :] = val
        recv_ref[my_pos, 1, :] = idx.astype(jnp.float32)

        pl.semaphore_wait(barrier_sem, N_DEV - 1)

        sends = []
        for o in range(1, N_DEV):
            peer = (my_pos - o) % N_DEV
            rdma = pltpu.make_async_remote_copy(
                src_ref=recv_ref.at[my_pos],
                dst_ref=recv_ref.at[my_pos],
                send_sem=send_sems.at[o],
                recv_sem=recv_sems.at[my_pos],
                device_id=(peer,),
                device_id_type=pl.DeviceIdType.MESH,
            )
            rdma.start()
            sends.append(rdma)

        for p in range(1, N_DEV):
            src = my_pos ^ p
            recv = pltpu.make_async_remote_copy(
                src_ref=recv_ref.at[src],
                dst_ref=recv_ref.at[src],
                send_sem=send_sems.at[p],
                recv_sem=recv_sems.at[src],
                device_id=(src,),
                device_id_type=pl.DeviceIdType.MESH,
            )
            recv.wait_recv()

        vals = recv_ref[:, 0, :]
        idxs = recv_ref[:, 1, :]
        m = jnp.max(vals, axis=0)
        gi = jnp.min(
            jnp.where(vals == m[None, :], idxs, float(N_DEV * m_per)), axis=0
        )
        out_ref[0, :] = m
        out_ref[1, :] = gi

        for rdma in sends:
            rdma.wait_send()

    return pl.pallas_call(
        body,
        out_shape=jax.ShapeDtypeStruct((2, n), jnp.float32),
        in_specs=[pl.BlockSpec(memory_space=pltpu.VMEM)],
        out_specs=pl.BlockSpec(memory_space=pltpu.VMEM),
        scratch_shapes=[
            pltpu.VMEM((N_DEV, 2, n), jnp.float32),
            pltpu.SemaphoreType.DMA((N_DEV,)),
            pltpu.SemaphoreType.DMA((N_DEV,)),
        ],
        compiler_params=pltpu.CompilerParams(collective_id=0),
    )(x)
